# baseline (speedup 1.0000x reference)
import functools

import jax
import jax.numpy as jnp
from jax import lax
from jax.experimental import pallas as pl
from jax.experimental.pallas import tpu as pltpu
from jax.experimental.pallas import tpu_sc as plsc

_NC = 2
_NS = 16
_NW = _NC * _NS
_CHUNK = 2000


def _proj_body(nodes_ref, wr_ref, ws_ref, glob_ref, wg_ref, b_ref,
               pr_ref, ps_ref, c_ref):
    n = nodes_ref[...]
    pr_ref[...] = jnp.dot(n, wr_ref[...], preferred_element_type=jnp.float32)
    ps_ref[...] = jnp.dot(n, ws_ref[...], preferred_element_type=jnp.float32)
    c_ref[...] = jnp.dot(glob_ref[...], wg_ref[...],
                         preferred_element_type=jnp.float32) + b_ref[...]


def _epart_body(e_ref, wbig_ref, cbig_ref, o_ref):
    o_ref[...] = jnp.dot(e_ref[...], wbig_ref[...],
                         preferred_element_type=jnp.float32) + cbig_ref[...]


def _sc_body(epw, nchunk, pack, pr_hbm, ps_hbm, recv_hbm, send_hbm,
             epart_hbm, out_hbm, ridx, sidx, rrows, srows, erows,
             sem_r, sem_s, sem_e):
    wid = lax.axis_index("s") * _NC + lax.axis_index("c")
    base = wid * epw
    crows = _CHUNK // pack
    d_out = 128 // pack

    def chunk_body(ci, carry):
        off = base + ci * _CHUNK
        roff = off // pack
        pltpu.sync_copy(recv_hbm.at[pl.ds(off, _CHUNK)], ridx)
        pltpu.sync_copy(send_hbm.at[pl.ds(off, _CHUNK)], sidx)
        cr = pltpu.async_copy(pr_hbm.at[ridx], rrows, sem_r)
        cs = pltpu.async_copy(ps_hbm.at[sidx], srows, sem_s)
        ce = pltpu.async_copy(epart_hbm.at[pl.ds(roff, crows)], erows, sem_e)
        cr.wait()
        cs.wait()
        ce.wait()

        def row_body(r, carry2):
            e = r * pack
            for k in range(pack):
                sl = pl.ds(k * d_out, d_out)
                erows[r, sl] = erows[r, sl] + rrows[e + k, :] + srows[e + k, :]
            return carry2

        lax.fori_loop(0, crows, row_body, 0)
        pltpu.sync_copy(erows, out_hbm.at[pl.ds(roff, crows)])
        return carry

    lax.fori_loop(0, nchunk, chunk_body, 0)


def kernel(edges, nodes, globals_, receivers, senders, W, b):
    E, d_edge = edges.shape
    N, d_node = nodes.shape
    d_out = W.shape[-1]
    f32 = jnp.float32
    pack = 128 // d_out

    we = W[:d_edge]
    wr = W[d_edge:d_edge + d_node]
    ws = W[d_edge + d_node:d_edge + 2 * d_node]
    wg = W[d_edge + 2 * d_node:]
    b2 = b.reshape(1, d_out)
    we_big = jnp.kron(jnp.eye(pack, dtype=f32), we)

    recv32 = receivers.astype(jnp.int32)
    send32 = senders.astype(jnp.int32)

    pr, ps, c = pl.pallas_call(
        _proj_body,
        out_shape=[jax.ShapeDtypeStruct((N, d_out), f32),
                   jax.ShapeDtypeStruct((N, d_out), f32),
                   jax.ShapeDtypeStruct((1, d_out), f32)],
    )(nodes, wr, ws, globals_, wg, b2)

    rows = E // pack
    cbig = jnp.tile(c, (1, pack))
    ef = edges.reshape(rows, 128)
    br = 4000
    epart = pl.pallas_call(
        _epart_body,
        grid=(rows // br,),
        in_specs=[pl.BlockSpec((br, 128), lambda i: (i, 0)),
                  pl.BlockSpec((128, 128), lambda i: (0, 0)),
                  pl.BlockSpec((1, 128), lambda i: (0, 0))],
        out_specs=pl.BlockSpec((br, 128), lambda i: (i, 0)),
        out_shape=jax.ShapeDtypeStruct((rows, 128), f32),
    )(ef, we_big, cbig)

    epw = E // _NW
    nchunk = epw // _CHUNK
    mesh = plsc.VectorSubcoreMesh(core_axis_name="c", subcore_axis_name="s")
    sc = pl.kernel(
        functools.partial(_sc_body, epw, nchunk, pack),
        mesh=mesh,
        compiler_params=pltpu.CompilerParams(use_tc_tiling_on_sc=False),
        out_type=jax.ShapeDtypeStruct((rows, 128), f32),
        scratch_types=[
            pltpu.VMEM((_CHUNK,), jnp.int32),
            pltpu.VMEM((_CHUNK,), jnp.int32),
            pltpu.VMEM((_CHUNK, d_out), f32),
            pltpu.VMEM((_CHUNK, d_out), f32),
            pltpu.VMEM((_CHUNK // pack, 128), f32),
            pltpu.SemaphoreType.DMA,
            pltpu.SemaphoreType.DMA,
            pltpu.SemaphoreType.DMA,
        ],
    )
    out = sc(pr, ps, recv32, send32, epart)
    return out.reshape(E, d_out)


# speedup vs baseline: 1.3701x; 1.3701x over previous
import functools

import jax
import jax.numpy as jnp
from jax import lax
from jax.experimental import pallas as pl
from jax.experimental.pallas import tpu as pltpu
from jax.experimental.pallas import tpu_sc as plsc

_NC = 2
_NS = 16
_NW = _NC * _NS
_CHUNK = 2000


def _proj_body(nodes_ref, wr_ref, ws_ref, glob_ref, wg_ref, b_ref,
               pr_ref, ps_ref, c_ref):
    n = nodes_ref[...]
    hi = jax.lax.Precision.HIGHEST
    pr_ref[...] = jnp.dot(n, wr_ref[...], precision=hi,
                          preferred_element_type=jnp.float32)
    ps_ref[...] = jnp.dot(n, ws_ref[...], precision=hi,
                          preferred_element_type=jnp.float32)
    c_ref[...] = jnp.dot(glob_ref[...], wg_ref[...], precision=hi,
                         preferred_element_type=jnp.float32) + b_ref[...]


def _sc_body(epw, nchunk, pack, d_out, pr_hbm, ps_hbm, recv_hbm, send_hbm,
             g_hbm, ridx, sidx, rrows, srows, sem_r, sem_s):
    wid = lax.axis_index("s") * _NC + lax.axis_index("c")
    base = wid * epw

    def chunk_body(ci, carry):
        off = base + ci * _CHUNK
        c = off // _CHUNK
        band = c % pack
        r0 = (c // pack) * _CHUNK
        pltpu.sync_copy(recv_hbm.at[pl.ds(off, _CHUNK)], ridx)
        pltpu.sync_copy(send_hbm.at[pl.ds(off, _CHUNK)], sidx)
        cr = pltpu.async_copy(pr_hbm.at[ridx], rrows, sem_r)
        cs = pltpu.async_copy(ps_hbm.at[sidx], srows, sem_s)
        cr.wait()
        cs.wait()

        def row_body(r, carry2):
            e = r * 4
            for k in range(4):
                rrows[e + k, :] = rrows[e + k, :] + srows[e + k, :]
            return carry2

        lax.fori_loop(0, _CHUNK // 4, row_body, 0)
        pltpu.sync_copy(
            rrows, g_hbm.at[pl.ds(r0, _CHUNK), pl.ds(band * d_out, d_out)])
        return carry


    lax.fori_loop(0, nchunk, chunk_body, 0)


def _combine_body(pack, et_ref, g_ref, wet_ref, ct_ref, eye_ref, ot_ref):
    hi = jax.lax.Precision.HIGHEST
    edge_term = jnp.dot(wet_ref[...], et_ref[...], precision=hi,
                        preferred_element_type=jnp.float32)
    d_out = 128 // pack
    ct = ct_ref[...]
    for k in range(pack):
        g_t = jax.lax.dot_general(
            eye_ref[...], g_ref[:, k * d_out:(k + 1) * d_out],
            (((1,), (1,)), ((), ())), precision=hi,
            preferred_element_type=jnp.float32)
        ot_ref[:, k * _CHUNK:(k + 1) * _CHUNK] = (
            edge_term[:, k * _CHUNK:(k + 1) * _CHUNK] + g_t + ct)


def kernel(edges, nodes, globals_, receivers, senders, W, b):
    E, d_edge = edges.shape
    N, d_node = nodes.shape
    d_out = W.shape[-1]
    f32 = jnp.float32
    pack = 128 // d_out

    we = W[:d_edge]
    wr = W[d_edge:d_edge + d_node]
    ws = W[d_edge + d_node:d_edge + 2 * d_node]
    wg = W[d_edge + 2 * d_node:]
    b2 = b.reshape(1, d_out)

    recv32 = receivers.astype(jnp.int32)
    send32 = senders.astype(jnp.int32)

    pr, ps, c = pl.pallas_call(
        _proj_body,
        out_shape=[jax.ShapeDtypeStruct((N, d_out), f32),
                   jax.ShapeDtypeStruct((N, d_out), f32),
                   jax.ShapeDtypeStruct((1, d_out), f32)],
    )(nodes, wr, ws, globals_, wg, b2)

    epw = E // _NW
    nchunk = epw // _CHUNK
    grows = E // pack
    mesh = plsc.VectorSubcoreMesh(core_axis_name="c", subcore_axis_name="s")
    sc = pl.kernel(
        functools.partial(_sc_body, epw, nchunk, pack, d_out),
        mesh=mesh,
        compiler_params=pltpu.CompilerParams(use_tc_tiling_on_sc=False),
        out_type=jax.ShapeDtypeStruct((grows, 128), f32),
        scratch_types=[
            pltpu.VMEM((_CHUNK,), jnp.int32),
            pltpu.VMEM((_CHUNK,), jnp.int32),
            pltpu.VMEM((_CHUNK, d_out), f32),
            pltpu.VMEM((_CHUNK, d_out), f32),
            pltpu.SemaphoreType.DMA,
            pltpu.SemaphoreType.DMA,
        ],
    )
    g = sc(pr, ps, recv32, send32)

    et = edges.T
    wet = we.T
    ct = c.T
    eye16 = jnp.eye(d_out, dtype=f32)
    eb = pack * _CHUNK
    out_t = pl.pallas_call(
        functools.partial(_combine_body, pack),
        grid=(E // eb,),
        in_specs=[
            pl.BlockSpec((d_out, eb), lambda i: (0, i)),
            pl.BlockSpec((_CHUNK, 128), lambda i: (i, 0)),
            pl.BlockSpec((d_out, d_out), lambda i: (0, 0)),
            pl.BlockSpec((d_out, 1), lambda i: (0, 0)),
            pl.BlockSpec((d_out, d_out), lambda i: (0, 0)),
        ],
        out_specs=pl.BlockSpec((d_out, eb), lambda i: (0, i)),
        out_shape=jax.ShapeDtypeStruct((d_out, E), f32),
    )(et, g, wet, ct, eye16)
    return out_t.T


# speedup vs baseline: 2.3856x; 1.7411x over previous
import functools

import jax
import jax.numpy as jnp
from jax import lax
from jax.experimental import pallas as pl
from jax.experimental.pallas import tpu as pltpu
from jax.experimental.pallas import tpu_sc as plsc

_NC = 2
_NS = 16
_NW = _NC * _NS
_CHUNK = 2000


def _proj_body(nodes_ref, wr_ref, ws_ref, glob_ref, wg_ref, b_ref,
               pr_ref, ps_ref, c_ref):
    n = nodes_ref[...]
    hi = jax.lax.Precision.HIGHEST
    pr_ref[...] = jnp.dot(n, wr_ref[...], precision=hi,
                          preferred_element_type=jnp.float32)
    ps_ref[...] = jnp.dot(n, ws_ref[...], precision=hi,
                          preferred_element_type=jnp.float32)
    c_ref[...] = jnp.dot(glob_ref[...], wg_ref[...], precision=hi,
                         preferred_element_type=jnp.float32) + b_ref[...]


def _sc_body(epw, nchunk, pack, d_out, pr_hbm, ps_hbm, recv_hbm, send_hbm,
             g_hbm, ridx, sidx, rrows, srows, sem_r, sem_s):
    wid = lax.axis_index("s") * _NC + lax.axis_index("c")
    base = wid * epw

    def chunk_body(ci, carry):
        off = base + ci * _CHUNK
        c = off // _CHUNK
        band = c % pack
        r0 = (c // pack) * _CHUNK
        pltpu.sync_copy(recv_hbm.at[pl.ds(off, _CHUNK)], ridx)
        pltpu.sync_copy(send_hbm.at[pl.ds(off, _CHUNK)], sidx)
        cr = pltpu.async_copy(pr_hbm.at[ridx], rrows, sem_r)
        cs = pltpu.async_copy(ps_hbm.at[sidx], srows, sem_s)
        cr.wait()
        cs.wait()

        def row_body(r, carry2):
            e = r * 4
            for k in range(4):
                rrows[e + k, :] = rrows[e + k, :] + srows[e + k, :]
            return carry2

        lax.fori_loop(0, _CHUNK // 4, row_body, 0)
        pltpu.sync_copy(
            rrows, g_hbm.at[pl.ds(r0, _CHUNK), pl.ds(band * d_out, d_out)])
        return carry


    lax.fori_loop(0, nchunk, chunk_body, 0)


def _combine_body(pack, et_ref, g_ref, wet_ref, ct_ref, eye_ref, ot_ref):
    hi = jax.lax.Precision.HIGHEST
    edge_term = jnp.dot(wet_ref[...], et_ref[...], precision=hi,
                        preferred_element_type=jnp.float32)
    d_out = 128 // pack
    ct = ct_ref[...]
    for k in range(pack):
        g_t = jax.lax.dot_general(
            eye_ref[...], g_ref[:, k * d_out:(k + 1) * d_out],
            (((1,), (1,)), ((), ())),
            preferred_element_type=jnp.float32)
        ot_ref[:, k * _CHUNK:(k + 1) * _CHUNK] = (
            edge_term[:, k * _CHUNK:(k + 1) * _CHUNK] + g_t + ct)


def kernel(edges, nodes, globals_, receivers, senders, W, b):
    E, d_edge = edges.shape
    N, d_node = nodes.shape
    d_out = W.shape[-1]
    f32 = jnp.float32
    pack = 128 // d_out

    we = W[:d_edge]
    wr = W[d_edge:d_edge + d_node]
    ws = W[d_edge + d_node:d_edge + 2 * d_node]
    wg = W[d_edge + 2 * d_node:]
    b2 = b.reshape(1, d_out)

    recv32 = receivers.astype(jnp.int32)
    send32 = senders.astype(jnp.int32)

    pr, ps, c = pl.pallas_call(
        _proj_body,
        out_shape=[jax.ShapeDtypeStruct((N, d_out), f32),
                   jax.ShapeDtypeStruct((N, d_out), f32),
                   jax.ShapeDtypeStruct((1, d_out), f32)],
    )(nodes, wr, ws, globals_, wg, b2)

    epw = E // _NW
    nchunk = epw // _CHUNK
    grows = E // pack
    mesh = plsc.VectorSubcoreMesh(core_axis_name="c", subcore_axis_name="s")
    sc = pl.kernel(
        functools.partial(_sc_body, epw, nchunk, pack, d_out),
        mesh=mesh,
        compiler_params=pltpu.CompilerParams(use_tc_tiling_on_sc=False),
        out_type=jax.ShapeDtypeStruct((grows, 128), f32),
        scratch_types=[
            pltpu.VMEM((_CHUNK,), jnp.int32),
            pltpu.VMEM((_CHUNK,), jnp.int32),
            pltpu.VMEM((_CHUNK, d_out), f32),
            pltpu.VMEM((_CHUNK, d_out), f32),
            pltpu.SemaphoreType.DMA,
            pltpu.SemaphoreType.DMA,
        ],
    )
    g = sc(pr, ps, recv32, send32)

    et = edges.T
    wet = we.T
    ct = c.T
    eye16 = jnp.eye(d_out, dtype=f32)
    eb = pack * _CHUNK
    out_t = pl.pallas_call(
        functools.partial(_combine_body, pack),
        grid=(E // eb,),
        in_specs=[
            pl.BlockSpec((d_out, eb), lambda i: (0, i)),
            pl.BlockSpec((_CHUNK, 128), lambda i: (i, 0)),
            pl.BlockSpec((d_out, d_out), lambda i: (0, 0)),
            pl.BlockSpec((d_out, 1), lambda i: (0, 0)),
            pl.BlockSpec((d_out, d_out), lambda i: (0, 0)),
        ],
        out_specs=pl.BlockSpec((d_out, eb), lambda i: (0, i)),
        out_shape=jax.ShapeDtypeStruct((d_out, E), f32),
    )(et, g, wet, ct, eye16)
    return out_t.T
